# baseline (speedup 1.0000x reference)
import functools

import numpy as np
import jax
import jax.numpy as jnp
from jax import lax
from jax.experimental import pallas as pl
from jax.experimental.pallas import tpu as pltpu
from jax.experimental.pallas import tpu_sc as plsc

D = 1024
H = 16
DH = 64
DFF = 4096
K_SEL = 1000
NPER = 5440
NTOK = 10880
PSEG = 1024
TP = 2 * PSEG
NV = NPER // 16
U_ROWS = 11520

_f32 = jnp.float32
_bf16 = jnp.bfloat16
_i32 = jnp.int32

_i_of = (np.arange(D) % DH) // 2
_fx = 100.0 ** (-np.arange(14) / 14)
_fy = 100.0 ** (-np.arange(14) / 14)
_fl = 10.0 ** (-np.arange(4) / 4)
_FX = np.where(_i_of < 14, _fx[np.minimum(_i_of, 13)], 0.0)
_FY = np.where((_i_of >= 14) & (_i_of < 28),
               _fy[np.clip(_i_of - 14, 0, 13)], 0.0)
_FL = np.where(_i_of >= 28, _fl[np.clip(_i_of - 28, 0, 3)], 0.0)
_FREQ = np.zeros((8, D), np.float32)
_FREQ[0] = _FX
_FREQ[1] = _FY
_FREQ[2] = _FL


def _sc_select_body(se_hbm, ss_hbm, px_hbm, py_hbm, lv_hbm,
                    idxg_hbm, idxs_hbm, pxo_hbm, pyo_hbm, lvo_hbm,
                    key_v, sub_v, px_v, py_v, lv_v,
                    og_v, os_v, opx_v, opy_v, olv_v):
    c = lax.axis_index("c")
    s = lax.axis_index("s")

    @pl.when(s == 0)
    def _():
        b = c
        seg = pl.ds(pl.multiple_of(b * NPER, 8), NPER)
        pltpu.sync_copy(se_hbm.at[seg], px_v)
        pltpu.sync_copy(ss_hbm.at[seg], py_v)

        zeros = jnp.zeros((16,), _i32)
        kvec = jnp.full((16,), K_SEL, _i32)
        himask = jnp.full((16,), -65536, _i32)

        def key_body(i, carry):
            sl = pl.ds(i * 16, 16)
            x = px_v[sl] + py_v[sl]
            bits = lax.bitcast_convert_type(x, _i32)
            key = jnp.where(bits < 0, bits ^ jnp.int32(0x7FFFFFFF), bits)
            key_v[sl] = key
            return carry
        lax.fori_loop(0, NV, key_body, 0)

        pltpu.sync_copy(px_hbm.at[seg], px_v)
        pltpu.sync_copy(py_hbm.at[seg], py_v)
        pltpu.sync_copy(lv_hbm.at[seg], lv_v)

        def count_full(pred):
            def body(i, acc):
                kv = key_v[pl.ds(i * 16, 16)]
                return acc + plsc.all_reduce_population_count(pred(kv))
            return lax.fori_loop(0, NV, body, zeros)

        def bisect(cnt_fn, lo0, hi0, target, iters):
            def body(_, lh):
                lo, hi = lh
                mid = (lo + hi + 1) >> 1
                ok = cnt_fn(mid) >= target
                return (jnp.where(ok, mid, lo), jnp.where(ok, hi, mid - 1))
            lo, _ = lax.fori_loop(0, iters, body, (lo0, hi0))
            return lo

        vhi = bisect(
            lambda v: count_full(lambda kv: (kv & himask) >= (v << 16)),
            jnp.full((16,), -32768, _i32), jnp.full((16,), 32767, _i32),
            kvec, 16)
        base = vhi << 16
        n_hi = count_full(lambda kv: (kv & himask) > base)

        def fill_body(i, carry):
            sub_v[pl.ds(i * 16, 16)] = base
            return carry
        lax.fori_loop(0, NV, fill_body, 0)

        def comp_body(i, cnt):
            kv = key_v[pl.ds(i * 16, 16)]
            m = (kv & himask) == base
            pos = cnt + plsc.cumsum(jnp.where(m, 1, 0)) - 1
            plsc.store_scatter(sub_v, [pos], kv, mask=m)
            return cnt + plsc.all_reduce_population_count(m)
        n_eq = lax.fori_loop(0, NV, comp_body, zeros)
        nv_sub = jnp.max((n_eq + 15) >> 4)

        def count_sub(t):
            def body(i, acc):
                sv = sub_v[pl.ds(i * 16, 16)]
                return acc + plsc.all_reduce_population_count(sv >= t)
            return lax.fori_loop(0, nv_sub, body, zeros)

        wlo = bisect(lambda w: count_sub(base + w),
                     zeros, jnp.full((16,), 65535, _i32),
                     kvec - n_hi, 16)
        tstar = base + wlo

        def init_body(i, carry):
            sl = pl.ds(i * 16, 16)
            og_v[sl] = zeros
            os_v[sl] = jnp.full((16,), NTOK, _i32)
            fz = jnp.zeros((16,), _f32)
            opx_v[sl] = fz
            opy_v[sl] = fz
            olv_v[sl] = fz
            return carry
        lax.fori_loop(0, PSEG // 16, init_body, 0)

        lanes = lax.iota(_i32, 16)

        def scatter_sel(i, pos, m):
            gidx = b * NPER + i * 16 + lanes
            plsc.store_scatter(og_v, [pos], gidx, mask=m)
            plsc.store_scatter(os_v, [pos], gidx, mask=m)
            sl = pl.ds(i * 16, 16)
            plsc.store_scatter(opx_v, [pos], px_v[sl], mask=m)
            plsc.store_scatter(opy_v, [pos], py_v[sl], mask=m)
            plsc.store_scatter(olv_v, [pos], lv_v[sl], mask=m)

        def passA(i, cnt):
            kv = key_v[pl.ds(i * 16, 16)]
            m = kv > tstar
            pos = cnt + plsc.cumsum(jnp.where(m, 1, 0)) - 1
            scatter_sel(i, pos, m)
            return cnt + plsc.all_reduce_population_count(m)
        cnt = lax.fori_loop(0, NV, passA, zeros)

        def passB(i, cnt):
            kv = key_v[pl.ds(i * 16, 16)]
            m = kv == tstar
            pos = cnt + plsc.cumsum(jnp.where(m, 1, 0)) - 1
            scatter_sel(i, pos, m & (pos < K_SEL))
            return cnt + plsc.all_reduce_population_count(m)
        lax.fori_loop(0, NV, passB, cnt)

        oseg = pl.ds(pl.multiple_of(b * PSEG, 8), PSEG)
        pltpu.sync_copy(og_v, idxg_hbm.at[oseg])
        pltpu.sync_copy(os_v, idxs_hbm.at[oseg])
        pltpu.sync_copy(opx_v, pxo_hbm.at[oseg])
        pltpu.sync_copy(opy_v, pyo_hbm.at[oseg])
        pltpu.sync_copy(olv_v, lvo_hbm.at[oseg])


def _sc_select(tes, tpss, px, py, lv):
    mesh = plsc.VectorSubcoreMesh(core_axis_name="c", subcore_axis_name="s")
    out_type = (
        jax.ShapeDtypeStruct((TP,), _i32),
        jax.ShapeDtypeStruct((TP,), _i32),
        jax.ShapeDtypeStruct((TP,), _f32),
        jax.ShapeDtypeStruct((TP,), _f32),
        jax.ShapeDtypeStruct((TP,), _f32),
    )
    scratch = [
        pltpu.VMEM((NPER,), _i32),
        pltpu.VMEM((NPER,), _i32),
        pltpu.VMEM((NPER,), _f32),
        pltpu.VMEM((NPER,), _f32),
        pltpu.VMEM((NPER,), _f32),
        pltpu.VMEM((PSEG,), _i32),
        pltpu.VMEM((PSEG,), _i32),
        pltpu.VMEM((PSEG,), _f32),
        pltpu.VMEM((PSEG,), _f32),
        pltpu.VMEM((PSEG,), _f32),
    ]
    fn = pl.kernel(_sc_select_body, out_type=out_type, mesh=mesh,
                   scratch_types=scratch,
                   compiler_params=pltpu.CompilerParams(
                       needs_layout_passes=False))
    return fn(tes, tpss, px, py, lv)


def _sc_gather_body(q_hbm, idx_hbm, x_hbm, idx_v, rows_v, sem):
    wid = lax.axis_index("s") * 2 + lax.axis_index("c")
    base = pl.multiple_of(wid * (TP // 32), 8)
    pltpu.sync_copy(idx_hbm.at[pl.ds(base, TP // 32)], idx_v)
    pltpu.async_copy(q_hbm.at[idx_v], rows_v, sem).wait()
    pltpu.sync_copy(rows_v, x_hbm.at[pl.ds(base, TP // 32)])


def _sc_gather(queries, idxg):
    mesh = plsc.VectorSubcoreMesh(core_axis_name="c", subcore_axis_name="s")
    fn = pl.kernel(
        _sc_gather_body,
        out_type=jax.ShapeDtypeStruct((TP, D), _f32),
        mesh=mesh,
        scratch_types=[
            pltpu.VMEM((TP // 32,), _i32),
            pltpu.VMEM((TP // 32, D), _f32),
            pltpu.SemaphoreType.DMA,
        ],
        compiler_params=pltpu.CompilerParams(needs_layout_passes=False))
    return fn(queries, idxg)


def _qkv_body(x_ref, px_ref, py_ref, lv_ref, fq_ref, g_ref, b_ref, w_ref,
              q_ref, k_ref, v_ref):
    xb = x_ref[...]
    mu = jnp.mean(xb, -1, keepdims=True)
    xc = xb - mu
    var = jnp.mean(xc * xc, -1, keepdims=True)
    h = xc * lax.rsqrt(var + 1e-5) * g_ref[...] + b_ref[...]
    qkv = jnp.dot(h.astype(_bf16), w_ref[...], preferred_element_type=_f32)
    ang = (px_ref[...] * fq_ref[0:1, :] + py_ref[...] * fq_ref[1:2, :]
           + lv_ref[...] * fq_ref[2:3, :])
    cc = jnp.cos(ang)
    ss = jnp.sin(ang)
    q_ref[...] = (qkv[:, 0:D] * cc + qkv[:, D:2 * D] * ss).astype(_bf16)
    k_ref[...] = (qkv[:, 2 * D:3 * D] * cc
                  + qkv[:, 3 * D:4 * D] * ss).astype(_bf16)
    v_ref[...] = qkv[:, 4 * D:5 * D].astype(_bf16)


def _qkv(x, pxs, pys, lvs, freq, g, b, wbig):
    rb = 512
    grid = (TP // rb,)
    return pl.pallas_call(
        _qkv_body,
        grid=grid,
        in_specs=[
            pl.BlockSpec((rb, D), lambda i: (i, 0)),
            pl.BlockSpec((rb, 1), lambda i: (i, 0)),
            pl.BlockSpec((rb, 1), lambda i: (i, 0)),
            pl.BlockSpec((rb, 1), lambda i: (i, 0)),
            pl.BlockSpec((8, D), lambda i: (0, 0)),
            pl.BlockSpec((1, D), lambda i: (0, 0)),
            pl.BlockSpec((1, D), lambda i: (0, 0)),
            pl.BlockSpec((D, 5 * D), lambda i: (0, 0)),
        ],
        out_specs=[
            pl.BlockSpec((rb, D), lambda i: (i, 0)),
            pl.BlockSpec((rb, D), lambda i: (i, 0)),
            pl.BlockSpec((rb, D), lambda i: (i, 0)),
        ],
        out_shape=[jax.ShapeDtypeStruct((TP, D), _bf16)] * 3,
    )(x, pxs, pys, lvs, freq, g, b, wbig)


def _attn_body(q_ref, k_ref, v_ref, o_ref):
    qh = q_ref[...].reshape(PSEG, DH)
    kh = k_ref[...].reshape(PSEG, DH)
    sc = lax.dot_general(qh, kh, (((1,), (1,)), ((), ())),
                         preferred_element_type=_f32)
    sc = sc * (1.0 / 8.0)
    col = lax.broadcasted_iota(_i32, (PSEG, PSEG), 1)
    sc = jnp.where(col < K_SEL, sc, -1e30)
    m = jnp.max(sc, -1, keepdims=True)
    p = jnp.exp(sc - m)
    p = p / jnp.sum(p, -1, keepdims=True)
    o_ref[...] = jnp.dot(p.astype(_bf16), v_ref[...].reshape(PSEG, DH),
                         preferred_element_type=_f32).astype(_bf16).reshape(
                             1, PSEG, DH)


def _attn(q3, k3, v3):
    return pl.pallas_call(
        _attn_body,
        grid=(2 * H,),
        in_specs=[
            pl.BlockSpec((1, PSEG, DH), lambda i: (i, 0, 0)),
            pl.BlockSpec((1, PSEG, DH), lambda i: (i, 0, 0)),
            pl.BlockSpec((1, PSEG, DH), lambda i: (i, 0, 0)),
        ],
        out_specs=pl.BlockSpec((1, PSEG, DH), lambda i: (i, 0, 0)),
        out_shape=jax.ShapeDtypeStruct((2 * H, PSEG, DH), _bf16),
    )(q3, k3, v3)


def _wo_body(o_ref, w_ref, x_ref, out_ref):
    out_ref[...] = (jnp.dot(o_ref[...], w_ref[...],
                            preferred_element_type=_f32) + x_ref[...])


def _wo(o, wo_bf, x):
    return pl.pallas_call(
        _wo_body,
        grid=(2,),
        in_specs=[
            pl.BlockSpec((PSEG, D), lambda i: (i, 0)),
            pl.BlockSpec((D, D), lambda i: (0, 0)),
            pl.BlockSpec((PSEG, D), lambda i: (i, 0)),
        ],
        out_specs=pl.BlockSpec((PSEG, D), lambda i: (i, 0)),
        out_shape=jax.ShapeDtypeStruct((TP, D), _f32),
    )(o, wo_bf, x)


def _copy_body(q_ref, o_ref):
    o_ref[...] = q_ref[...]


def _copy_u(queries):
    rb = 640
    return pl.pallas_call(
        _copy_body,
        grid=(U_ROWS // rb,),
        in_specs=[pl.BlockSpec((rb, D), lambda i: (jnp.minimum(i, 16), 0))],
        out_specs=pl.BlockSpec((rb, D), lambda i: (i, 0)),
        out_shape=jax.ShapeDtypeStruct((U_ROWS, D), _f32),
    )(queries)


def _scatter_body(idx_ref, a_ref, u_ref, o_ref):
    o_ref[...] = a_ref[...]


def _scatter_u(idxs, attn_out, u0):
    grid_spec = pltpu.PrefetchScalarGridSpec(
        num_scalar_prefetch=1,
        grid=(TP,),
        in_specs=[
            pl.BlockSpec((1, 1, D), lambda i, idx_ref: (i, 0, 0)),
            pl.BlockSpec((1, 1, D), lambda i, idx_ref: (0, 0, 0)),
        ],
        out_specs=pl.BlockSpec((1, 1, D),
                               lambda i, idx_ref: (idx_ref[i], 0, 0)),
    )
    out = pl.pallas_call(
        _scatter_body,
        grid_spec=grid_spec,
        out_shape=jax.ShapeDtypeStruct((U_ROWS, 1, D), _f32),
        input_output_aliases={2: 0},
    )(idxs, attn_out.reshape(TP, 1, D), u0.reshape(U_ROWS, 1, D))
    return out.reshape(U_ROWS, D)


def _ffn_body(u_ref, g_ref, bb_ref, w1_ref, b1_ref, w2_ref, b2_ref, o_ref):
    ub = u_ref[...]
    mu = jnp.mean(ub, -1, keepdims=True)
    uc = ub - mu
    var = jnp.mean(uc * uc, -1, keepdims=True)
    h = (uc * lax.rsqrt(var + 1e-5) * g_ref[...] + bb_ref[...]).astype(_bf16)
    acc = jnp.zeros(ub.shape, _f32)
    for j in range(4):
        a = jnp.dot(h, w1_ref[:, j * D:(j + 1) * D],
                    preferred_element_type=_f32) + b1_ref[0:1, j * D:(j + 1) * D]
        a = 0.5 * a * (1.0 + lax.erf(a * 0.7071067811865476))
        acc = acc + jnp.dot(a.astype(_bf16), w2_ref[j * D:(j + 1) * D, :],
                            preferred_element_type=_f32)
    o_ref[...] = ub + acc + b2_ref[...]


def _ffn(u, g, b, w1_bf, b1, w2_bf, b2):
    rb = 640
    return pl.pallas_call(
        _ffn_body,
        grid=(NTOK // rb,),
        in_specs=[
            pl.BlockSpec((rb, D), lambda i: (i, 0)),
            pl.BlockSpec((1, D), lambda i: (0, 0)),
            pl.BlockSpec((1, D), lambda i: (0, 0)),
            pl.BlockSpec((D, DFF), lambda i: (0, 0)),
            pl.BlockSpec((1, DFF), lambda i: (0, 0)),
            pl.BlockSpec((DFF, D), lambda i: (0, 0)),
            pl.BlockSpec((1, D), lambda i: (0, 0)),
        ],
        out_specs=pl.BlockSpec((rb, D), lambda i: (i, 0)),
        out_shape=jax.ShapeDtypeStruct((NTOK, D), _f32),
    )(u, g, b, w1_bf, b1, w2_bf, b2)


def _pair_swap(w):
    w4 = w.reshape(D, D // 2, 2)
    return jnp.stack([-w4[:, :, 1], w4[:, :, 0]], axis=-1).reshape(D, D)


def kernel(queries, query_batch_offsets, token_predicted_salience_score,
           query_spatial_indices, stacked_feature_maps, level_spatial_shapes,
           token_electron_scores, ln1_g, ln1_b, Wqkv, Wo, ln2_g, ln2_b,
           W1, b1, W2, b2):
    px = query_spatial_indices[1].astype(_f32)
    py = query_spatial_indices[2].astype(_f32)
    lv = query_spatial_indices[3].astype(_f32)

    idxg, idxs, pxs, pys, lvs = _sc_select(
        token_electron_scores, token_predicted_salience_score, px, py, lv)
    x = _sc_gather(queries, idxg)

    wq = Wqkv[:, :D]
    wk = Wqkv[:, D:2 * D]
    wv = Wqkv[:, 2 * D:]
    wbig = jnp.concatenate(
        [wq, _pair_swap(wq), wk, _pair_swap(wk), wv], axis=1).astype(_bf16)
    freq = jnp.asarray(_FREQ)

    q, k, v = _qkv(x, pxs.reshape(TP, 1), pys.reshape(TP, 1),
                   lvs.reshape(TP, 1), freq,
                   ln1_g.reshape(1, D), ln1_b.reshape(1, D), wbig)

    def to_heads(a):
        return a.reshape(2, PSEG, H, DH).transpose(0, 2, 1, 3).reshape(
            2 * H, PSEG, DH)

    o3 = _attn(to_heads(q), to_heads(k), to_heads(v))
    o = o3.reshape(2, H, PSEG, DH).transpose(0, 2, 1, 3).reshape(TP, D)
    attn_out = _wo(o, Wo.astype(_bf16), x)

    u0 = _copy_u(queries)
    u = _scatter_u(idxs, attn_out, u0)

    out = _ffn(u, ln2_g.reshape(1, D), ln2_b.reshape(1, D),
               W1.astype(_bf16), b1.reshape(1, DFF),
               W2.astype(_bf16), b2.reshape(1, D))
    return out
